# Initial kernel scaffold; baseline (speedup 1.0000x reference)
import jax
import jax.numpy as jnp
from jax.experimental import pallas as pl


def kernel(input1, input2):
    raise NotImplementedError("write your pallas kernel here")


# speedup vs baseline: 1.7083x; 1.7083x over previous
import functools

import jax
import jax.numpy as jnp
from jax.experimental import pallas as pl


def _nnd_body(x_ref, y_ref, d1_ref, d2_ref, *, tn, mc):
    n = pl.program_id(1)
    x = x_ref[0]
    y = y_ref[0]
    m_total = y.shape[1]
    x0 = x[:, 0:1]
    x1 = x[:, 1:2]
    x2 = x[:, 2:3]
    rm = None
    for j in range(m_total // mc):
        ys = y[:, j * mc:(j + 1) * mc]
        d0 = x0 - ys[0:1, :]
        acc = d0 * d0
        d1 = x1 - ys[1:2, :]
        acc = acc + d1 * d1
        d2 = x2 - ys[2:3, :]
        acc = acc + d2 * d2
        rmj = jnp.min(acc, axis=1)
        rm = rmj if rm is None else jnp.minimum(rm, rmj)
        cmj = jnp.min(acc, axis=0)
        sl = pl.ds(j * mc, mc)
        prev = jnp.where(n == 0, jnp.full((mc,), jnp.inf, acc.dtype),
                         d2_ref[0, 0, sl])
        d2_ref[0, 0, sl] = jnp.minimum(prev, cmj)
    d1_ref[0, 0, pl.ds(n * tn, tn)] = rm


@jax.jit
def kernel(input1, input2):
    b, n, _ = input1.shape
    m = input2.shape[1]
    tn = 128
    mc = 256
    yt = input2.transpose(0, 2, 1)
    d1, d2 = pl.pallas_call(
        functools.partial(_nnd_body, tn=tn, mc=mc),
        grid=(b, n // tn),
        in_specs=[
            pl.BlockSpec((1, tn, 3), lambda b_, n_: (b_, n_, 0)),
            pl.BlockSpec((1, 3, m), lambda b_, n_: (b_, 0, 0)),
        ],
        out_specs=[
            pl.BlockSpec((1, 1, n), lambda b_, n_: (b_, 0, 0)),
            pl.BlockSpec((1, 1, m), lambda b_, n_: (b_, 0, 0)),
        ],
        out_shape=[
            jax.ShapeDtypeStruct((b, 1, n), input1.dtype),
            jax.ShapeDtypeStruct((b, 1, m), input1.dtype),
        ],
    )(input1, yt)
    return d1.reshape(b, n), d2.reshape(b, m)
